# Initial kernel scaffold; baseline (speedup 1.0000x reference)
import jax
import jax.numpy as jnp
from jax.experimental import pallas as pl


def kernel(x, edge_index, W1, b1, Wq1, bq1, Wk1, bk1, Wv1, bv1, g1, be1, a1, W2, b2, g2, be2, a2, W3, b3, g3, be3, a3, Wh, bh):
    raise NotImplementedError("write your pallas kernel here")


# speedup vs baseline: 5.6685x; 5.6685x over previous
import functools

import jax
import jax.numpy as jnp
from jax import lax
from jax.experimental import pallas as pl
from jax.experimental.pallas import tpu as pltpu
from jax.experimental.pallas import tpu_sc as plsc

NC = 2
NS = 16
CHUNK = 80
HN = 5120
HNS = HN + 16
EPS = 1e-5


def _sc_mesh():
    return plsc.VectorSubcoreMesh(core_axis_name="c", subcore_axis_name="s")


def _idx_body(dst_ref, out_ref):
    d = dst_ref[...]
    out_ref[0] = jnp.where(d < HN, d, HN)
    out_ref[1] = jnp.where(d >= HN, d - HN, HN)


def _local_dst(dst):
    e = dst.shape[0]
    return pl.pallas_call(
        _idx_body,
        out_shape=jax.ShapeDtypeStruct((NC, e // 128, 128), jnp.int32),
    )(dst.reshape(e // 128, 128)).reshape(NC * e)


def _deghist_body(dstt_ref, c_ref):
    nt = dstt_ref.shape[1] // 128
    io80 = lax.broadcasted_iota(jnp.int32, (1, 80), 1)
    io128 = lax.broadcasted_iota(jnp.int32, (1, 128), 1)

    def step(j, acc):
        base = pl.multiple_of(j * 128, 128)
        tile = dstt_ref[:, pl.ds(base, 128)]
        hi = jax.lax.shift_right_logical(tile, 7)
        lo = jax.lax.bitwise_and(tile, 127)
        for k in range(128):
            u = (hi[:, k:k + 1] == io80).astype(jnp.float32)
            v = (lo[:, k:k + 1] == io128).astype(jnp.float32)
            acc = acc + lax.dot_general(u, v, (((0,), (0,)), ((), ())),
                                        preferred_element_type=jnp.float32)
        return acc

    c_ref[...] = lax.fori_loop(0, nt, step,
                               jnp.zeros((80, 128), jnp.float32))


def _deg_hist(dst):
    e = dst.shape[0]
    dstt = dst.reshape(e // 128, 128).T
    ncol = dstt.shape[1]
    pad = (-ncol) % 128
    if pad:
        dstt = jnp.concatenate(
            [dstt, jnp.full((128, pad), 80 * 128 - 1, jnp.int32)], axis=1)
    c = pl.pallas_call(
        _deghist_body,
        out_shape=jax.ShapeDtypeStruct((80, 128), jnp.float32),
    )(dstt)
    return c.reshape(80 * 128, 1)


def _scatter_partials(hp, src, ldst, zeros_c):
    n, d = hp.shape
    e = src.shape[0]
    ec = e // NS
    nit = ec // CHUNK
    rpt = HN // NS
    rptz = HNS // NS

    @functools.partial(
        pl.kernel,
        out_type=jax.ShapeDtypeStruct((NC, NS, rpt, d), jnp.float32),
        mesh=_sc_mesh(),
        scratch_types=[
            pltpu.VMEM((CHUNK,), jnp.int32),
            pltpu.VMEM((CHUNK,), jnp.int32),
            pltpu.VMEM((CHUNK, d), jnp.float32),
            pltpu.VMEM_SHARED((HNS, d), jnp.float32),
            pltpu.SemaphoreType.DMA,
        ],
    )
    def scat_k(hp_h, src_h, ldst_h, zeros_h, out_h,
               sidx, didx, rows, shared, sem):
        c = lax.axis_index("c")
        s = lax.axis_index("s")
        pltpu.sync_copy(zeros_h, shared.at[pl.ds(s * rptz, rptz)])
        plsc.subcore_barrier()

        eb = s * ec
        ebd = c * e + s * ec

        def step(j, carry):
            pltpu.sync_copy(src_h.at[pl.ds(eb + j * CHUNK, CHUNK)], sidx)
            pltpu.sync_copy(ldst_h.at[pl.ds(ebd + j * CHUNK, CHUNK)], didx)
            pltpu.async_copy(hp_h.at[sidx], rows, sem).wait()
            pltpu.sync_copy(rows, shared.at[didx], add=True)
            return carry

        lax.fori_loop(0, nit, step, 0)
        plsc.subcore_barrier()
        pltpu.sync_copy(shared.at[pl.ds(s * rpt, rpt)], out_h.at[c, s])

    return scat_k(hp, src, ldst, zeros_c).reshape(NC, HN, d)


def _gnorm(t, g, be, a):
    mean = jnp.mean(t, axis=0, keepdims=True)
    h = t - mean * a
    var = jnp.mean(h * h, axis=0, keepdims=True)
    return g * h / jnp.sqrt(var + EPS) + be


def _dot(a, b):
    return jnp.dot(a, b, preferred_element_type=jnp.float32)


def _pre_body(x_ref, w1_ref, wq_ref, bq_ref, wk_ref, bk_ref, wv_ref, bv_ref,
              h1_ref, q_ref, kv_ref, ksum_ref):
    x = x_ref[...]
    h1_ref[...] = _dot(x, w1_ref[...])
    q = jnp.maximum(_dot(x, wq_ref[...]) + bq_ref[...], 0.0)
    k = jnp.maximum(_dot(x, wk_ref[...]) + bk_ref[...], 0.0)
    v = _dot(x, wv_ref[...]) + bv_ref[...]
    q_ref[...] = q
    kv_ref[...] = lax.dot_general(k, v, (((0,), (0,)), ((), ())),
                                  preferred_element_type=jnp.float32)
    ksum_ref[...] = jnp.sum(k, axis=0, keepdims=True)


def _scale_body(pdeg_ref, h1_ref, dinv_ref, h1p_ref):
    n = h1_ref.shape[0]
    deg = pdeg_ref[...][:n] + 1.0
    dinv = lax.rsqrt(jnp.maximum(deg, 1.0))
    dinv_ref[...] = dinv
    h1p_ref[...] = h1_ref[...] * dinv


def _gather_s(s_ref, n):
    v = s_ref[...]
    return jnp.concatenate([v[0], v[1]], axis=0)[:n]


def _mid1_body(s_ref, hp_ref, dinv_ref, b_ref, q_ref, kv_ref, ksum_ref,
               g_ref, be_ref, a_ref, w2_ref, x1_ref, h2p_ref):
    n = x1_ref.shape[0]
    dinv = dinv_ref[...]
    gcn = (_gather_s(s_ref, n) + hp_ref[...]) * dinv + b_ref[...]
    q = q_ref[...]
    num = _dot(q, kv_ref[...])
    den = jnp.sum(q * ksum_ref[...], axis=1, keepdims=True) + 1e-6
    att = num / den
    x1 = jnp.maximum(_gnorm(gcn + att, g_ref[...], be_ref[...], a_ref[...]),
                     0.0)
    x1_ref[...] = x1
    h2p_ref[...] = _dot(x1, w2_ref[...]) * dinv


def _mid2_body(s_ref, hp_ref, dinv_ref, b_ref, g_ref, be_ref, a_ref,
               xprev_ref, w_ref, xo_ref, hpo_ref):
    n = xo_ref.shape[0]
    dinv = dinv_ref[...]
    gcn = (_gather_s(s_ref, n) + hp_ref[...]) * dinv + b_ref[...]
    xo = jnp.maximum(_gnorm(gcn, g_ref[...], be_ref[...], a_ref[...]),
                     0.0) + xprev_ref[...]
    xo_ref[...] = xo
    hpo_ref[...] = _dot(xo, w_ref[...]) * dinv


def _fin_body(s_ref, hp_ref, dinv_ref, b_ref, g_ref, be_ref, a_ref,
              x2_ref, x_ref, wa_ref, wb_ref, bh_ref, out_ref):
    n = out_ref.shape[0]
    dinv = dinv_ref[...]
    gcn = (_gather_s(s_ref, n) + hp_ref[...]) * dinv + b_ref[...]
    x3 = jnp.maximum(_gnorm(gcn, g_ref[...], be_ref[...], a_ref[...]),
                     0.0) + x2_ref[...]
    out_ref[...] = _dot(x_ref[...], wa_ref[...]) + _dot(x3, wb_ref[...]) \
        + bh_ref[...]


def _f32(shape):
    return jax.ShapeDtypeStruct(shape, jnp.float32)


def kernel(x, edge_index, W1, b1, Wq1, bq1, Wk1, bk1, Wv1, bv1, g1, be1, a1,
           W2, b2, g2, be2, a2, W3, b3, g3, be3, a3, Wh, bh):
    n, in_c = x.shape
    hid = W1.shape[1]
    out_c = Wh.shape[1]
    e = edge_index.shape[1]
    src = edge_index[0]
    dst = edge_index[1]

    r = lambda v: v.reshape(1, -1)
    WhA = Wh[:in_c]
    WhB = Wh[in_c:]

    zerosd = jnp.zeros((HNS // NS, hid), jnp.float32)

    ldst = _local_dst(dst)
    pdeg = _deg_hist(dst)

    h1, Q, KV, Ksum = pl.pallas_call(
        _pre_body,
        out_shape=[_f32((n, hid)), _f32((n, hid)), _f32((hid, hid)),
                   _f32((1, hid))],
    )(x, W1, Wq1, r(bq1), Wk1, r(bk1), Wv1, r(bv1))

    dinv, h1p = pl.pallas_call(
        _scale_body,
        out_shape=[_f32((n, 1)), _f32((n, hid))],
    )(pdeg, h1)

    S1 = _scatter_partials(h1p, src, ldst, zerosd)
    x1, h2p = pl.pallas_call(
        _mid1_body,
        out_shape=[_f32((n, hid)), _f32((n, hid))],
    )(S1, h1p, dinv, r(b1), Q, KV, Ksum, r(g1), r(be1), r(a1), W2)

    S2 = _scatter_partials(h2p, src, ldst, zerosd)
    x2, h3p = pl.pallas_call(
        _mid2_body,
        out_shape=[_f32((n, hid)), _f32((n, hid))],
    )(S2, h2p, dinv, r(b2), r(g2), r(be2), r(a2), x1, W3)

    S3 = _scatter_partials(h3p, src, ldst, zerosd)
    out = pl.pallas_call(
        _fin_body,
        out_shape=_f32((n, out_c)),
    )(S3, h3p, dinv, r(b3), r(g3), r(be3), r(a3), x2, x, WhA, WhB, r(bh))
    return out


# speedup vs baseline: 9.0211x; 1.5914x over previous
import functools

import jax
import jax.numpy as jnp
from jax import lax
from jax.experimental import pallas as pl
from jax.experimental.pallas import tpu as pltpu
from jax.experimental.pallas import tpu_sc as plsc

NC = 2
NS = 16
CHUNK = 80
HN = 5120
HNS = HN + 16
EPS = 1e-5


def _sc_mesh():
    return plsc.VectorSubcoreMesh(core_axis_name="c", subcore_axis_name="s")


def _idx_body(dst_ref, out_ref):
    d = dst_ref[...]
    out_ref[0] = jnp.where(d < HN, d, HN)
    out_ref[1] = jnp.where(d >= HN, d - HN, HN)


def _local_dst(dst):
    e = dst.shape[0]
    return pl.pallas_call(
        _idx_body,
        out_shape=jax.ShapeDtypeStruct((NC, e // 128, 128), jnp.int32),
    )(dst.reshape(e // 128, 128)).reshape(NC * e)


def _deghist_body(dstt_ref, c_ref):
    nt = dstt_ref.shape[1] // 128
    io80 = lax.broadcasted_iota(jnp.int32, (1, 80), 1)
    io128 = lax.broadcasted_iota(jnp.int32, (1, 128), 1)

    def step(j, acc):
        base = pl.multiple_of(j * 128, 128)
        tile = dstt_ref[:, pl.ds(base, 128)]
        hi = jax.lax.shift_right_logical(tile, 7)
        lo = jax.lax.bitwise_and(tile, 127)
        for k in range(128):
            u = (hi[:, k:k + 1] == io80).astype(jnp.float32)
            v = (lo[:, k:k + 1] == io128).astype(jnp.float32)
            acc = acc + lax.dot_general(u, v, (((0,), (0,)), ((), ())),
                                        preferred_element_type=jnp.float32)
        return acc

    c_ref[...] = lax.fori_loop(0, nt, step,
                               jnp.zeros((80, 128), jnp.float32))


def _deg_hist(dst):
    e = dst.shape[0]
    dstt = dst.reshape(e // 128, 128).T
    ncol = dstt.shape[1]
    pad = (-ncol) % 128
    if pad:
        dstt = jnp.concatenate(
            [dstt, jnp.full((128, pad), 80 * 128 - 1, jnp.int32)], axis=1)
    c = pl.pallas_call(
        _deghist_body,
        out_shape=jax.ShapeDtypeStruct((80, 128), jnp.float32),
    )(dstt)
    return c.reshape(80 * 128, 1)


CH = 100
BK = 5


def _scatter_partials(hp, src3, ldst3, zeros_c, zdrain):
    n, d = hp.shape
    nrow, bk, ch = src3.shape
    ec_rows = nrow // NS
    rpt = HN // NS
    rptz = HNS // NS

    @functools.partial(
        pl.kernel,
        out_type=jax.ShapeDtypeStruct((NC, NS, rpt, d), jnp.float32),
        mesh=_sc_mesh(),
        scratch_types=[
            pltpu.VMEM((BK, CH), jnp.int32),
            pltpu.VMEM((BK, CH), jnp.int32),
            pltpu.VMEM((BK, CH, d), jnp.float32),
            pltpu.VMEM_SHARED((HNS, d), jnp.float32),
            pltpu.SemaphoreType.DMA,
            pltpu.SemaphoreType.DMA,
        ],
    )
    def scat_k(hp_h, src_h, ldst_h, zeros_h, zdrain_h, out_h,
               sblk, dblk, rows, shared, gsem, ssem):
        c = lax.axis_index("c")
        s = lax.axis_index("s")
        pltpu.sync_copy(zeros_h, shared.at[pl.ds(s * rptz, rptz)])
        plsc.subcore_barrier()

        rb = s * ec_rows
        rbd = c * nrow + s * ec_rows

        def step(j, carry):
            @pl.when(j > 0)
            def _():
                pltpu.make_async_copy(zdrain_h, rows, ssem).wait()

            pltpu.sync_copy(src_h.at[rb + j], sblk)
            pltpu.sync_copy(ldst_h.at[rbd + j], dblk)
            descs = [pltpu.async_copy(hp_h.at[sblk.at[b]], rows.at[b], gsem)
                     for b in range(BK)]
            for dd in descs:
                dd.wait()
            for b in range(BK):
                pltpu.async_copy(rows.at[b], shared.at[dblk.at[b]], ssem,
                                 add=True)
            return carry

        lax.fori_loop(0, ec_rows, step, 0)
        pltpu.make_async_copy(zdrain_h, rows, ssem).wait()
        plsc.subcore_barrier()
        pltpu.sync_copy(shared.at[pl.ds(s * rpt, rpt)], out_h.at[c, s])

    return scat_k(hp, src3, ldst3, zeros_c, zdrain).reshape(NC, HN, d)


def _gnorm(t, g, be, a):
    mean = jnp.mean(t, axis=0, keepdims=True)
    h = t - mean * a
    var = jnp.mean(h * h, axis=0, keepdims=True)
    return g * h / jnp.sqrt(var + EPS) + be


def _dot(a, b):
    return jnp.dot(a, b, preferred_element_type=jnp.float32)


def _pre_body(x_ref, w1_ref, wq_ref, bq_ref, wk_ref, bk_ref, wv_ref, bv_ref,
              h1_ref, q_ref, kv_ref, ksum_ref):
    x = x_ref[...]
    h1_ref[...] = _dot(x, w1_ref[...])
    q = jnp.maximum(_dot(x, wq_ref[...]) + bq_ref[...], 0.0)
    k = jnp.maximum(_dot(x, wk_ref[...]) + bk_ref[...], 0.0)
    v = _dot(x, wv_ref[...]) + bv_ref[...]
    q_ref[...] = q
    kv_ref[...] = lax.dot_general(k, v, (((0,), (0,)), ((), ())),
                                  preferred_element_type=jnp.float32)
    ksum_ref[...] = jnp.sum(k, axis=0, keepdims=True)


def _scale_body(pdeg_ref, h1_ref, dinv_ref, h1p_ref):
    n = h1_ref.shape[0]
    deg = pdeg_ref[...][:n] + 1.0
    dinv = lax.rsqrt(jnp.maximum(deg, 1.0))
    dinv_ref[...] = dinv
    h1p_ref[...] = h1_ref[...] * dinv


def _gather_s(s_ref, n):
    v = s_ref[...]
    return jnp.concatenate([v[0], v[1]], axis=0)[:n]


def _mid1_body(s_ref, hp_ref, dinv_ref, b_ref, q_ref, kv_ref, ksum_ref,
               g_ref, be_ref, a_ref, w2_ref, x1_ref, h2p_ref):
    n = x1_ref.shape[0]
    dinv = dinv_ref[...]
    gcn = (_gather_s(s_ref, n) + hp_ref[...]) * dinv + b_ref[...]
    q = q_ref[...]
    num = _dot(q, kv_ref[...])
    den = jnp.sum(q * ksum_ref[...], axis=1, keepdims=True) + 1e-6
    att = num / den
    x1 = jnp.maximum(_gnorm(gcn + att, g_ref[...], be_ref[...], a_ref[...]),
                     0.0)
    x1_ref[...] = x1
    h2p_ref[...] = _dot(x1, w2_ref[...]) * dinv


def _mid2_body(s_ref, hp_ref, dinv_ref, b_ref, g_ref, be_ref, a_ref,
               xprev_ref, w_ref, xo_ref, hpo_ref):
    n = xo_ref.shape[0]
    dinv = dinv_ref[...]
    gcn = (_gather_s(s_ref, n) + hp_ref[...]) * dinv + b_ref[...]
    xo = jnp.maximum(_gnorm(gcn, g_ref[...], be_ref[...], a_ref[...]),
                     0.0) + xprev_ref[...]
    xo_ref[...] = xo
    hpo_ref[...] = _dot(xo, w_ref[...]) * dinv


def _fin_body(s_ref, hp_ref, dinv_ref, b_ref, g_ref, be_ref, a_ref,
              x2_ref, x_ref, wa_ref, wb_ref, bh_ref, out_ref):
    n = out_ref.shape[0]
    dinv = dinv_ref[...]
    gcn = (_gather_s(s_ref, n) + hp_ref[...]) * dinv + b_ref[...]
    x3 = jnp.maximum(_gnorm(gcn, g_ref[...], be_ref[...], a_ref[...]),
                     0.0) + x2_ref[...]
    out_ref[...] = _dot(x_ref[...], wa_ref[...]) + _dot(x3, wb_ref[...]) \
        + bh_ref[...]


def _f32(shape):
    return jax.ShapeDtypeStruct(shape, jnp.float32)


def kernel(x, edge_index, W1, b1, Wq1, bq1, Wk1, bk1, Wv1, bv1, g1, be1, a1,
           W2, b2, g2, be2, a2, W3, b3, g3, be3, a3, Wh, bh):
    n, in_c = x.shape
    hid = W1.shape[1]
    out_c = Wh.shape[1]
    e = edge_index.shape[1]
    src = edge_index[0]
    dst = edge_index[1]

    r = lambda v: v.reshape(1, -1)
    WhA = Wh[:in_c]
    WhB = Wh[in_c:]

    zerosd = jnp.zeros((HNS // NS, hid), jnp.float32)
    zdrain = jnp.zeros((BK, CH, hid), jnp.float32)

    src3 = src.reshape(e // (BK * CH), BK, CH)
    ldst3 = _local_dst(dst).reshape(NC * e // (BK * CH), BK, CH)
    pdeg = _deg_hist(dst)

    h1, Q, KV, Ksum = pl.pallas_call(
        _pre_body,
        out_shape=[_f32((n, hid)), _f32((n, hid)), _f32((hid, hid)),
                   _f32((1, hid))],
    )(x, W1, Wq1, r(bq1), Wk1, r(bk1), Wv1, r(bv1))

    dinv, h1p = pl.pallas_call(
        _scale_body,
        out_shape=[_f32((n, 1)), _f32((n, hid))],
    )(pdeg, h1)

    S1 = _scatter_partials(h1p, src3, ldst3, zerosd, zdrain)
    x1, h2p = pl.pallas_call(
        _mid1_body,
        out_shape=[_f32((n, hid)), _f32((n, hid))],
    )(S1, h1p, dinv, r(b1), Q, KV, Ksum, r(g1), r(be1), r(a1), W2)

    S2 = _scatter_partials(h2p, src3, ldst3, zerosd, zdrain)
    x2, h3p = pl.pallas_call(
        _mid2_body,
        out_shape=[_f32((n, hid)), _f32((n, hid))],
    )(S2, h2p, dinv, r(b2), r(g2), r(be2), r(a2), x1, W3)

    S3 = _scatter_partials(h3p, src3, ldst3, zerosd, zdrain)
    out = pl.pallas_call(
        _fin_body,
        out_shape=_f32((n, out_c)),
    )(S3, h3p, dinv, r(b3), r(g3), r(be3), r(a3), x2, x, WhA, WhB, r(bh))
    return out


# speedup vs baseline: 9.2145x; 1.0214x over previous
import functools

import jax
import jax.numpy as jnp
from jax import lax
from jax.experimental import pallas as pl
from jax.experimental.pallas import tpu as pltpu
from jax.experimental.pallas import tpu_sc as plsc

NC = 2
NS = 16
CHUNK = 80
HN = 5120
HNS = HN + 16
EPS = 1e-5


def _sc_mesh():
    return plsc.VectorSubcoreMesh(core_axis_name="c", subcore_axis_name="s")


def _idx_body(dst_ref, out_ref):
    d = dst_ref[...]
    out_ref[0] = jnp.where(d < HN, d, HN)
    out_ref[1] = jnp.where(d >= HN, d - HN, HN)


def _local_dst(dst):
    e = dst.shape[0]
    return pl.pallas_call(
        _idx_body,
        out_shape=jax.ShapeDtypeStruct((NC, e // 128, 128), jnp.int32),
    )(dst.reshape(e // 128, 128)).reshape(NC * e)


def _deghist_body(dstt_ref, c_ref):
    nt = dstt_ref.shape[1] // 128
    io80 = lax.broadcasted_iota(jnp.int32, (1, 80), 1)
    io128 = lax.broadcasted_iota(jnp.int32, (1, 128), 1)

    def step(j, acc):
        base = pl.multiple_of(j * 128, 128)
        tile = dstt_ref[:, pl.ds(base, 128)]
        hi = jax.lax.shift_right_logical(tile, 7)
        lo = jax.lax.bitwise_and(tile, 127)
        for k in range(128):
            u = (hi[:, k:k + 1] == io80).astype(jnp.float32)
            v = (lo[:, k:k + 1] == io128).astype(jnp.float32)
            acc = acc + lax.dot_general(u, v, (((0,), (0,)), ((), ())),
                                        preferred_element_type=jnp.float32)
        return acc

    c_ref[...] = lax.fori_loop(0, nt, step,
                               jnp.zeros((80, 128), jnp.float32))


def _deg_hist(dst):
    e = dst.shape[0]
    dstt = dst.reshape(e // 128, 128).T
    ncol = dstt.shape[1]
    pad = (-ncol) % 128
    if pad:
        dstt = jnp.concatenate(
            [dstt, jnp.full((128, pad), 80 * 128 - 1, jnp.int32)], axis=1)
    c = pl.pallas_call(
        _deghist_body,
        out_shape=jax.ShapeDtypeStruct((80, 128), jnp.float32),
    )(dstt)
    return c.reshape(80 * 128, 1)


CH = 50
BK = 8


def _scatter_partials(hp, src3, ldst3, zeros_c, zdrain):
    n, d = hp.shape
    nrow, bk, ch = src3.shape
    ec_rows = nrow // NS
    rpt = HN // NS
    rptz = HNS // NS

    @functools.partial(
        pl.kernel,
        out_type=jax.ShapeDtypeStruct((NC, NS, rpt, d), jnp.float32),
        mesh=_sc_mesh(),
        scratch_types=[
            pltpu.VMEM((BK, CH), jnp.int32),
            pltpu.VMEM((BK, CH), jnp.int32),
            pltpu.VMEM((BK, CH, d), jnp.float32),
            pltpu.VMEM_SHARED((HNS, d), jnp.float32),
            pltpu.SemaphoreType.DMA,
            pltpu.SemaphoreType.DMA,
        ],
    )
    def scat_k(hp_h, src_h, ldst_h, zeros_h, zdrain_h, out_h,
               sblk, dblk, rows, shared, gsem, ssem):
        c = lax.axis_index("c")
        s = lax.axis_index("s")
        pltpu.sync_copy(zeros_h, shared.at[pl.ds(s * rptz, rptz)])
        plsc.subcore_barrier()

        rb = s * ec_rows
        rbd = c * nrow + s * ec_rows

        def step(j, carry):
            @pl.when(j > 0)
            def _():
                pltpu.make_async_copy(zdrain_h, rows, ssem).wait()

            pltpu.sync_copy(src_h.at[rb + j], sblk)
            pltpu.sync_copy(ldst_h.at[rbd + j], dblk)
            half = BK // 2
            ga = [pltpu.async_copy(hp_h.at[sblk.at[b]], rows.at[b], gsem)
                  for b in range(half)]
            for dd in ga:
                dd.wait()
            for b in range(half):
                pltpu.async_copy(rows.at[b], shared.at[dblk.at[b]], ssem,
                                 add=True)
            gb = [pltpu.async_copy(hp_h.at[sblk.at[b]], rows.at[b], gsem)
                  for b in range(half, BK)]
            for dd in gb:
                dd.wait()
            for b in range(half, BK):
                pltpu.async_copy(rows.at[b], shared.at[dblk.at[b]], ssem,
                                 add=True)
            return carry

        lax.fori_loop(0, ec_rows, step, 0)
        pltpu.make_async_copy(zdrain_h, rows, ssem).wait()
        plsc.subcore_barrier()
        pltpu.sync_copy(shared.at[pl.ds(s * rpt, rpt)], out_h.at[c, s])

    return scat_k(hp, src3, ldst3, zeros_c, zdrain).reshape(NC, HN, d)


def _gnorm(t, g, be, a):
    mean = jnp.mean(t, axis=0, keepdims=True)
    h = t - mean * a
    var = jnp.mean(h * h, axis=0, keepdims=True)
    return g * h / jnp.sqrt(var + EPS) + be


def _dot(a, b):
    return jnp.dot(a, b, preferred_element_type=jnp.float32)


def _pre_body(x_ref, w1_ref, wq_ref, bq_ref, wk_ref, bk_ref, wv_ref, bv_ref,
              h1_ref, q_ref, kv_ref, ksum_ref):
    x = x_ref[...]
    h1_ref[...] = _dot(x, w1_ref[...])
    q = jnp.maximum(_dot(x, wq_ref[...]) + bq_ref[...], 0.0)
    k = jnp.maximum(_dot(x, wk_ref[...]) + bk_ref[...], 0.0)
    v = _dot(x, wv_ref[...]) + bv_ref[...]
    q_ref[...] = q
    kv_ref[...] = lax.dot_general(k, v, (((0,), (0,)), ((), ())),
                                  preferred_element_type=jnp.float32)
    ksum_ref[...] = jnp.sum(k, axis=0, keepdims=True)


def _scale_body(pdeg_ref, h1_ref, dinv_ref, h1p_ref):
    n = h1_ref.shape[0]
    deg = pdeg_ref[...][:n] + 1.0
    dinv = lax.rsqrt(jnp.maximum(deg, 1.0))
    dinv_ref[...] = dinv
    h1p_ref[...] = h1_ref[...] * dinv


def _gather_s(s_ref, n):
    v = s_ref[...]
    return jnp.concatenate([v[0], v[1]], axis=0)[:n]


def _mid1_body(s_ref, hp_ref, dinv_ref, b_ref, q_ref, kv_ref, ksum_ref,
               g_ref, be_ref, a_ref, w2_ref, x1_ref, h2p_ref):
    n = x1_ref.shape[0]
    dinv = dinv_ref[...]
    gcn = (_gather_s(s_ref, n) + hp_ref[...]) * dinv + b_ref[...]
    q = q_ref[...]
    num = _dot(q, kv_ref[...])
    den = jnp.sum(q * ksum_ref[...], axis=1, keepdims=True) + 1e-6
    att = num / den
    x1 = jnp.maximum(_gnorm(gcn + att, g_ref[...], be_ref[...], a_ref[...]),
                     0.0)
    x1_ref[...] = x1
    h2p_ref[...] = _dot(x1, w2_ref[...]) * dinv


def _mid2_body(s_ref, hp_ref, dinv_ref, b_ref, g_ref, be_ref, a_ref,
               xprev_ref, w_ref, xo_ref, hpo_ref):
    n = xo_ref.shape[0]
    dinv = dinv_ref[...]
    gcn = (_gather_s(s_ref, n) + hp_ref[...]) * dinv + b_ref[...]
    xo = jnp.maximum(_gnorm(gcn, g_ref[...], be_ref[...], a_ref[...]),
                     0.0) + xprev_ref[...]
    xo_ref[...] = xo
    hpo_ref[...] = _dot(xo, w_ref[...]) * dinv


def _fin_body(s_ref, hp_ref, dinv_ref, b_ref, g_ref, be_ref, a_ref,
              x2_ref, x_ref, wa_ref, wb_ref, bh_ref, out_ref):
    n = out_ref.shape[0]
    dinv = dinv_ref[...]
    gcn = (_gather_s(s_ref, n) + hp_ref[...]) * dinv + b_ref[...]
    x3 = jnp.maximum(_gnorm(gcn, g_ref[...], be_ref[...], a_ref[...]),
                     0.0) + x2_ref[...]
    out_ref[...] = _dot(x_ref[...], wa_ref[...]) + _dot(x3, wb_ref[...]) \
        + bh_ref[...]


def _f32(shape):
    return jax.ShapeDtypeStruct(shape, jnp.float32)


def kernel(x, edge_index, W1, b1, Wq1, bq1, Wk1, bk1, Wv1, bv1, g1, be1, a1,
           W2, b2, g2, be2, a2, W3, b3, g3, be3, a3, Wh, bh):
    n, in_c = x.shape
    hid = W1.shape[1]
    out_c = Wh.shape[1]
    e = edge_index.shape[1]
    src = edge_index[0]
    dst = edge_index[1]

    r = lambda v: v.reshape(1, -1)
    WhA = Wh[:in_c]
    WhB = Wh[in_c:]

    zerosd = jnp.zeros((HNS // NS, hid), jnp.float32)
    zdrain = jnp.zeros((BK, CH, hid), jnp.float32)

    src3 = src.reshape(e // (BK * CH), BK, CH)
    ldst3 = _local_dst(dst).reshape(NC * e // (BK * CH), BK, CH)
    pdeg = _deg_hist(dst)

    h1, Q, KV, Ksum = pl.pallas_call(
        _pre_body,
        out_shape=[_f32((n, hid)), _f32((n, hid)), _f32((hid, hid)),
                   _f32((1, hid))],
    )(x, W1, Wq1, r(bq1), Wk1, r(bk1), Wv1, r(bv1))

    dinv, h1p = pl.pallas_call(
        _scale_body,
        out_shape=[_f32((n, 1)), _f32((n, hid))],
    )(pdeg, h1)

    S1 = _scatter_partials(h1p, src3, ldst3, zerosd, zdrain)
    x1, h2p = pl.pallas_call(
        _mid1_body,
        out_shape=[_f32((n, hid)), _f32((n, hid))],
    )(S1, h1p, dinv, r(b1), Q, KV, Ksum, r(g1), r(be1), r(a1), W2)

    S2 = _scatter_partials(h2p, src3, ldst3, zerosd, zdrain)
    x2, h3p = pl.pallas_call(
        _mid2_body,
        out_shape=[_f32((n, hid)), _f32((n, hid))],
    )(S2, h2p, dinv, r(b2), r(g2), r(be2), r(a2), x1, W3)

    S3 = _scatter_partials(h3p, src3, ldst3, zerosd, zdrain)
    out = pl.pallas_call(
        _fin_body,
        out_shape=_f32((n, out_c)),
    )(S3, h3p, dinv, r(b3), r(g3), r(be3), r(a3), x2, x, WhA, WhB, r(bh))
    return out


# speedup vs baseline: 9.7119x; 1.0540x over previous
import functools

import jax
import jax.numpy as jnp
from jax import lax
from jax.experimental import pallas as pl
from jax.experimental.pallas import tpu as pltpu
from jax.experimental.pallas import tpu_sc as plsc

NC = 2
NS = 16
CHUNK = 80
HN = 5120
HNS = HN + 16
EPS = 1e-5


def _sc_mesh():
    return plsc.VectorSubcoreMesh(core_axis_name="c", subcore_axis_name="s")


def _idx_body(dst_ref, out_ref):
    d = dst_ref[...]
    out_ref[0] = jnp.where(d < HN, d, HN)
    out_ref[1] = jnp.where(d >= HN, d - HN, HN)


def _local_dst(dst):
    e = dst.shape[0]
    return pl.pallas_call(
        _idx_body,
        out_shape=jax.ShapeDtypeStruct((NC, e // 128, 128), jnp.int32),
    )(dst.reshape(e // 128, 128)).reshape(NC * e)


def _deghist_body(dstt_ref, c_ref):
    nt = dstt_ref.shape[1] // 128
    io80 = lax.broadcasted_iota(jnp.int32, (1, 80), 1)
    io128 = lax.broadcasted_iota(jnp.int32, (1, 128), 1)

    def step(j, acc):
        base = pl.multiple_of(j * 128, 128)
        tile = dstt_ref[:, pl.ds(base, 128)]
        hi = jax.lax.shift_right_logical(tile, 7)
        lo = jax.lax.bitwise_and(tile, 127)
        for k in range(128):
            u = (hi[:, k:k + 1] == io80).astype(jnp.float32)
            v = (lo[:, k:k + 1] == io128).astype(jnp.float32)
            acc = acc + lax.dot_general(u, v, (((0,), (0,)), ((), ())),
                                        preferred_element_type=jnp.float32)
        return acc

    c_ref[...] = lax.fori_loop(0, nt, step,
                               jnp.zeros((80, 128), jnp.float32))


def _deg_hist(dst):
    e = dst.shape[0]
    dstt = dst.reshape(e // 128, 128).T
    ncol = dstt.shape[1]
    pad = (-ncol) % 128
    if pad:
        dstt = jnp.concatenate(
            [dstt, jnp.full((128, pad), 80 * 128 - 1, jnp.int32)], axis=1)
    c = pl.pallas_call(
        _deghist_body,
        out_shape=jax.ShapeDtypeStruct((80, 128), jnp.float32),
    )(dstt)
    return c.reshape(80 * 128, 1)


CH = 50
BK = 8


def _scatter_partials(hp, src3, ldst3, zeros_c, zdrain):
    n, d = hp.shape
    nrow, bk, ch = src3.shape
    ec_rows = nrow // NS
    rpt = HN // NS
    rptz = HNS // NS

    @functools.partial(
        pl.kernel,
        out_type=jax.ShapeDtypeStruct((NC, NS, rpt, d), jnp.float32),
        mesh=_sc_mesh(),
        scratch_types=[
            pltpu.VMEM((BK, CH), jnp.int32),
            pltpu.VMEM((BK, CH), jnp.int32),
            pltpu.VMEM((BK, CH, d), jnp.float32),
            pltpu.VMEM_SHARED((HNS, d), jnp.float32),
            pltpu.SemaphoreType.DMA,
            pltpu.SemaphoreType.DMA,
        ],
    )
    def scat_k(hp_h, src_h, ldst_h, zeros_h, zdrain_h, out_h,
               sblk, dblk, rows, shared, gsem, ssem):
        c = lax.axis_index("c")
        s = lax.axis_index("s")
        pltpu.sync_copy(zeros_h, shared.at[pl.ds(s * rptz, rptz)])
        plsc.subcore_barrier()

        rb = s * ec_rows
        rbd = c * nrow + s * ec_rows

        def step(j, carry):
            @pl.when(j > 0)
            def _():
                pltpu.make_async_copy(zdrain_h, rows, ssem).wait()

            pltpu.sync_copy(src_h.at[rb + j], sblk)
            pltpu.sync_copy(ldst_h.at[rbd + j], dblk)
            ga = [pltpu.async_copy(hp_h.at[sblk.at[b]], rows.at[b], gsem)
                  for b in range(BK)]
            for b in range(BK):
                ga[b].wait()
                pltpu.async_copy(rows.at[b], shared.at[dblk.at[b]], ssem,
                                 add=True)
            return carry

        lax.fori_loop(0, ec_rows, step, 0)
        pltpu.make_async_copy(zdrain_h, rows, ssem).wait()
        plsc.subcore_barrier()
        pltpu.sync_copy(shared.at[pl.ds(s * rpt, rpt)], out_h.at[c, s])

    return scat_k(hp, src3, ldst3, zeros_c, zdrain).reshape(NC, HN, d)


def _gnorm(t, g, be, a):
    mean = jnp.mean(t, axis=0, keepdims=True)
    h = t - mean * a
    var = jnp.mean(h * h, axis=0, keepdims=True)
    return g * h / jnp.sqrt(var + EPS) + be


def _dot(a, b):
    return jnp.dot(a, b, preferred_element_type=jnp.float32)


def _pre_body(x_ref, w1_ref, wq_ref, bq_ref, wk_ref, bk_ref, wv_ref, bv_ref,
              h1_ref, q_ref, kv_ref, ksum_ref):
    x = x_ref[...]
    h1_ref[...] = _dot(x, w1_ref[...])
    q = jnp.maximum(_dot(x, wq_ref[...]) + bq_ref[...], 0.0)
    k = jnp.maximum(_dot(x, wk_ref[...]) + bk_ref[...], 0.0)
    v = _dot(x, wv_ref[...]) + bv_ref[...]
    q_ref[...] = q
    kv_ref[...] = lax.dot_general(k, v, (((0,), (0,)), ((), ())),
                                  preferred_element_type=jnp.float32)
    ksum_ref[...] = jnp.sum(k, axis=0, keepdims=True)


def _scale_body(pdeg_ref, h1_ref, dinv_ref, h1p_ref):
    n = h1_ref.shape[0]
    deg = pdeg_ref[...][:n] + 1.0
    dinv = lax.rsqrt(jnp.maximum(deg, 1.0))
    dinv_ref[...] = dinv
    h1p_ref[...] = h1_ref[...] * dinv


def _gather_s(s_ref, n):
    v = s_ref[...]
    return jnp.concatenate([v[0], v[1]], axis=0)[:n]


def _mid1_body(s_ref, hp_ref, dinv_ref, b_ref, q_ref, kv_ref, ksum_ref,
               g_ref, be_ref, a_ref, w2_ref, x1_ref, h2p_ref):
    n = x1_ref.shape[0]
    dinv = dinv_ref[...]
    gcn = (_gather_s(s_ref, n) + hp_ref[...]) * dinv + b_ref[...]
    q = q_ref[...]
    num = _dot(q, kv_ref[...])
    den = jnp.sum(q * ksum_ref[...], axis=1, keepdims=True) + 1e-6
    att = num / den
    x1 = jnp.maximum(_gnorm(gcn + att, g_ref[...], be_ref[...], a_ref[...]),
                     0.0)
    x1_ref[...] = x1
    h2p_ref[...] = _dot(x1, w2_ref[...]) * dinv


def _mid2_body(s_ref, hp_ref, dinv_ref, b_ref, g_ref, be_ref, a_ref,
               xprev_ref, w_ref, xo_ref, hpo_ref):
    n = xo_ref.shape[0]
    dinv = dinv_ref[...]
    gcn = (_gather_s(s_ref, n) + hp_ref[...]) * dinv + b_ref[...]
    xo = jnp.maximum(_gnorm(gcn, g_ref[...], be_ref[...], a_ref[...]),
                     0.0) + xprev_ref[...]
    xo_ref[...] = xo
    hpo_ref[...] = _dot(xo, w_ref[...]) * dinv


def _fin_body(s_ref, hp_ref, dinv_ref, b_ref, g_ref, be_ref, a_ref,
              x2_ref, x_ref, wa_ref, wb_ref, bh_ref, out_ref):
    n = out_ref.shape[0]
    dinv = dinv_ref[...]
    gcn = (_gather_s(s_ref, n) + hp_ref[...]) * dinv + b_ref[...]
    x3 = jnp.maximum(_gnorm(gcn, g_ref[...], be_ref[...], a_ref[...]),
                     0.0) + x2_ref[...]
    out_ref[...] = _dot(x_ref[...], wa_ref[...]) + _dot(x3, wb_ref[...]) \
        + bh_ref[...]


def _f32(shape):
    return jax.ShapeDtypeStruct(shape, jnp.float32)


def kernel(x, edge_index, W1, b1, Wq1, bq1, Wk1, bk1, Wv1, bv1, g1, be1, a1,
           W2, b2, g2, be2, a2, W3, b3, g3, be3, a3, Wh, bh):
    n, in_c = x.shape
    hid = W1.shape[1]
    out_c = Wh.shape[1]
    e = edge_index.shape[1]
    src = edge_index[0]
    dst = edge_index[1]

    r = lambda v: v.reshape(1, -1)
    WhA = Wh[:in_c]
    WhB = Wh[in_c:]

    zerosd = jnp.zeros((HNS // NS, hid), jnp.float32)
    zdrain = jnp.zeros((BK, CH, hid), jnp.float32)

    src3 = src.reshape(e // (BK * CH), BK, CH)
    ldst3 = _local_dst(dst).reshape(NC * e // (BK * CH), BK, CH)
    pdeg = _deg_hist(dst)

    h1, Q, KV, Ksum = pl.pallas_call(
        _pre_body,
        out_shape=[_f32((n, hid)), _f32((n, hid)), _f32((hid, hid)),
                   _f32((1, hid))],
    )(x, W1, Wq1, r(bq1), Wk1, r(bk1), Wv1, r(bv1))

    dinv, h1p = pl.pallas_call(
        _scale_body,
        out_shape=[_f32((n, 1)), _f32((n, hid))],
    )(pdeg, h1)

    S1 = _scatter_partials(h1p, src3, ldst3, zerosd, zdrain)
    x1, h2p = pl.pallas_call(
        _mid1_body,
        out_shape=[_f32((n, hid)), _f32((n, hid))],
    )(S1, h1p, dinv, r(b1), Q, KV, Ksum, r(g1), r(be1), r(a1), W2)

    S2 = _scatter_partials(h2p, src3, ldst3, zerosd, zdrain)
    x2, h3p = pl.pallas_call(
        _mid2_body,
        out_shape=[_f32((n, hid)), _f32((n, hid))],
    )(S2, h2p, dinv, r(b2), r(g2), r(be2), r(a2), x1, W3)

    S3 = _scatter_partials(h3p, src3, ldst3, zerosd, zdrain)
    out = pl.pallas_call(
        _fin_body,
        out_shape=_f32((n, out_c)),
    )(S3, h3p, dinv, r(b3), r(g3), r(be3), r(a3), x2, x, WhA, WhB, r(bh))
    return out


# speedup vs baseline: 9.8035x; 1.0094x over previous
import functools

import jax
import jax.numpy as jnp
from jax import lax
from jax.experimental import pallas as pl
from jax.experimental.pallas import tpu as pltpu
from jax.experimental.pallas import tpu_sc as plsc

NC = 2
NS = 16
CHUNK = 80
HN = 5120
HNS = HN + 16
EPS = 1e-5


def _sc_mesh():
    return plsc.VectorSubcoreMesh(core_axis_name="c", subcore_axis_name="s")


def _idx_body(dst_ref, out_ref):
    d = dst_ref[...]
    out_ref[0] = jnp.where(d < HN, d, HN)
    out_ref[1] = jnp.where(d >= HN, d - HN, HN)


def _local_dst(dst):
    e = dst.shape[0]
    return pl.pallas_call(
        _idx_body,
        out_shape=jax.ShapeDtypeStruct((NC, e // 128, 128), jnp.int32),
    )(dst.reshape(e // 128, 128)).reshape(NC * e)


def _deghist_body(dstt_ref, c_ref):
    nt = dstt_ref.shape[1] // 128
    io80 = lax.broadcasted_iota(jnp.int32, (1, 80), 1)
    io128 = lax.broadcasted_iota(jnp.int32, (1, 128), 1)

    def step(j, acc):
        base = pl.multiple_of(j * 128, 128)
        tile = dstt_ref[:, pl.ds(base, 128)]
        hi = jax.lax.shift_right_logical(tile, 7)
        lo = jax.lax.bitwise_and(tile, 127)
        for k in range(128):
            u = (hi[:, k:k + 1] == io80).astype(jnp.float32)
            v = (lo[:, k:k + 1] == io128).astype(jnp.float32)
            acc = acc + lax.dot_general(u, v, (((0,), (0,)), ((), ())),
                                        preferred_element_type=jnp.float32)
        return acc

    c_ref[...] = lax.fori_loop(0, nt, step,
                               jnp.zeros((80, 128), jnp.float32))


def _deg_hist(dst):
    e = dst.shape[0]
    dstt = dst.reshape(e // 128, 128).T
    ncol = dstt.shape[1]
    pad = (-ncol) % 128
    if pad:
        dstt = jnp.concatenate(
            [dstt, jnp.full((128, pad), 80 * 128 - 1, jnp.int32)], axis=1)
    c = pl.pallas_call(
        _deghist_body,
        out_shape=jax.ShapeDtypeStruct((80, 128), jnp.float32),
    )(dstt)
    return c.reshape(80 * 128, 1)


CH = 100
BK = 4


def _scatter_partials(hp, src3, ldst3, zeros_c, zdrain):
    n, d = hp.shape
    nrow, bk, ch = src3.shape
    ec_rows = nrow // NS
    rpt = HN // NS
    rptz = HNS // NS

    @functools.partial(
        pl.kernel,
        out_type=jax.ShapeDtypeStruct((NC, NS, rpt, d), jnp.float32),
        mesh=_sc_mesh(),
        scratch_types=[
            pltpu.VMEM((BK, CH), jnp.int32),
            pltpu.VMEM((BK, CH), jnp.int32),
            pltpu.VMEM((BK, CH, d), jnp.float32),
            pltpu.VMEM_SHARED((HNS, d), jnp.float32),
            pltpu.SemaphoreType.DMA,
            pltpu.SemaphoreType.DMA,
        ],
    )
    def scat_k(hp_h, src_h, ldst_h, zeros_h, zdrain_h, out_h,
               sblk, dblk, rows, shared, gsem, ssem):
        c = lax.axis_index("c")
        s = lax.axis_index("s")
        pltpu.sync_copy(zeros_h, shared.at[pl.ds(s * rptz, rptz)])
        plsc.subcore_barrier()

        rb = s * ec_rows
        rbd = c * nrow + s * ec_rows

        def step(j, carry):
            @pl.when(j > 0)
            def _():
                pltpu.make_async_copy(zdrain_h, rows, ssem).wait()

            pltpu.sync_copy(src_h.at[rb + j], sblk)
            pltpu.sync_copy(ldst_h.at[rbd + j], dblk)
            ga = [pltpu.async_copy(hp_h.at[sblk.at[b]], rows.at[b], gsem)
                  for b in range(BK)]
            for b in range(BK):
                ga[b].wait()
                pltpu.async_copy(rows.at[b], shared.at[dblk.at[b]], ssem,
                                 add=True)
            return carry

        lax.fori_loop(0, ec_rows, step, 0)
        pltpu.make_async_copy(zdrain_h, rows, ssem).wait()
        plsc.subcore_barrier()
        pltpu.sync_copy(shared.at[pl.ds(s * rpt, rpt)], out_h.at[c, s])

    return scat_k(hp, src3, ldst3, zeros_c, zdrain).reshape(NC, HN, d)


def _gnorm(t, g, be, a):
    mean = jnp.mean(t, axis=0, keepdims=True)
    h = t - mean * a
    var = jnp.mean(h * h, axis=0, keepdims=True)
    return g * h / jnp.sqrt(var + EPS) + be


def _dot(a, b):
    return jnp.dot(a, b, preferred_element_type=jnp.float32)


def _pre_body(x_ref, w1_ref, wq_ref, bq_ref, wk_ref, bk_ref, wv_ref, bv_ref,
              h1_ref, q_ref, kv_ref, ksum_ref):
    x = x_ref[...]
    h1_ref[...] = _dot(x, w1_ref[...])
    q = jnp.maximum(_dot(x, wq_ref[...]) + bq_ref[...], 0.0)
    k = jnp.maximum(_dot(x, wk_ref[...]) + bk_ref[...], 0.0)
    v = _dot(x, wv_ref[...]) + bv_ref[...]
    q_ref[...] = q
    kv_ref[...] = lax.dot_general(k, v, (((0,), (0,)), ((), ())),
                                  preferred_element_type=jnp.float32)
    ksum_ref[...] = jnp.sum(k, axis=0, keepdims=True)


def _scale_body(pdeg_ref, h1_ref, dinv_ref, h1p_ref):
    n = h1_ref.shape[0]
    deg = pdeg_ref[...][:n] + 1.0
    dinv = lax.rsqrt(jnp.maximum(deg, 1.0))
    dinv_ref[...] = dinv
    h1p_ref[...] = h1_ref[...] * dinv


def _gather_s(s_ref, n):
    v = s_ref[...]
    return jnp.concatenate([v[0], v[1]], axis=0)[:n]


def _mid1_body(s_ref, hp_ref, dinv_ref, b_ref, q_ref, kv_ref, ksum_ref,
               g_ref, be_ref, a_ref, w2_ref, x1_ref, h2p_ref):
    n = x1_ref.shape[0]
    dinv = dinv_ref[...]
    gcn = (_gather_s(s_ref, n) + hp_ref[...]) * dinv + b_ref[...]
    q = q_ref[...]
    num = _dot(q, kv_ref[...])
    den = jnp.sum(q * ksum_ref[...], axis=1, keepdims=True) + 1e-6
    att = num / den
    x1 = jnp.maximum(_gnorm(gcn + att, g_ref[...], be_ref[...], a_ref[...]),
                     0.0)
    x1_ref[...] = x1
    h2p_ref[...] = _dot(x1, w2_ref[...]) * dinv


def _mid2_body(s_ref, hp_ref, dinv_ref, b_ref, g_ref, be_ref, a_ref,
               xprev_ref, w_ref, xo_ref, hpo_ref):
    n = xo_ref.shape[0]
    dinv = dinv_ref[...]
    gcn = (_gather_s(s_ref, n) + hp_ref[...]) * dinv + b_ref[...]
    xo = jnp.maximum(_gnorm(gcn, g_ref[...], be_ref[...], a_ref[...]),
                     0.0) + xprev_ref[...]
    xo_ref[...] = xo
    hpo_ref[...] = _dot(xo, w_ref[...]) * dinv


def _fin_body(s_ref, hp_ref, dinv_ref, b_ref, g_ref, be_ref, a_ref,
              x2_ref, x_ref, wa_ref, wb_ref, bh_ref, out_ref):
    n = out_ref.shape[0]
    dinv = dinv_ref[...]
    gcn = (_gather_s(s_ref, n) + hp_ref[...]) * dinv + b_ref[...]
    x3 = jnp.maximum(_gnorm(gcn, g_ref[...], be_ref[...], a_ref[...]),
                     0.0) + x2_ref[...]
    out_ref[...] = _dot(x_ref[...], wa_ref[...]) + _dot(x3, wb_ref[...]) \
        + bh_ref[...]


def _f32(shape):
    return jax.ShapeDtypeStruct(shape, jnp.float32)


def kernel(x, edge_index, W1, b1, Wq1, bq1, Wk1, bk1, Wv1, bv1, g1, be1, a1,
           W2, b2, g2, be2, a2, W3, b3, g3, be3, a3, Wh, bh):
    n, in_c = x.shape
    hid = W1.shape[1]
    out_c = Wh.shape[1]
    e = edge_index.shape[1]
    src = edge_index[0]
    dst = edge_index[1]

    r = lambda v: v.reshape(1, -1)
    WhA = Wh[:in_c]
    WhB = Wh[in_c:]

    zerosd = jnp.zeros((HNS // NS, hid), jnp.float32)
    zdrain = jnp.zeros((BK, CH, hid), jnp.float32)

    src3 = src.reshape(e // (BK * CH), BK, CH)
    ldst3 = _local_dst(dst).reshape(NC * e // (BK * CH), BK, CH)
    pdeg = _deg_hist(dst)

    h1, Q, KV, Ksum = pl.pallas_call(
        _pre_body,
        out_shape=[_f32((n, hid)), _f32((n, hid)), _f32((hid, hid)),
                   _f32((1, hid))],
    )(x, W1, Wq1, r(bq1), Wk1, r(bk1), Wv1, r(bv1))

    dinv, h1p = pl.pallas_call(
        _scale_body,
        out_shape=[_f32((n, 1)), _f32((n, hid))],
    )(pdeg, h1)

    S1 = _scatter_partials(h1p, src3, ldst3, zerosd, zdrain)
    x1, h2p = pl.pallas_call(
        _mid1_body,
        out_shape=[_f32((n, hid)), _f32((n, hid))],
    )(S1, h1p, dinv, r(b1), Q, KV, Ksum, r(g1), r(be1), r(a1), W2)

    S2 = _scatter_partials(h2p, src3, ldst3, zerosd, zdrain)
    x2, h3p = pl.pallas_call(
        _mid2_body,
        out_shape=[_f32((n, hid)), _f32((n, hid))],
    )(S2, h2p, dinv, r(b2), r(g2), r(be2), r(a2), x1, W3)

    S3 = _scatter_partials(h3p, src3, ldst3, zerosd, zdrain)
    out = pl.pallas_call(
        _fin_body,
        out_shape=_f32((n, out_c)),
    )(S3, h3p, dinv, r(b3), r(g3), r(be3), r(a3), x2, x, WhA, WhB, r(bh))
    return out
